# Initial kernel scaffold; baseline (speedup 1.0000x reference)
import jax
import jax.numpy as jnp
from jax.experimental import pallas as pl


def kernel(batch, protos):
    raise NotImplementedError("write your pallas kernel here")


# speedup vs baseline: 1.1539x; 1.1539x over previous
import functools

import jax
import jax.numpy as jnp
from jax import lax
from jax.experimental import pallas as pl
from jax.experimental.pallas import tpu as pltpu
from jax.experimental.pallas import tpu_sc as plsc


M_TILE = 1024
K_TILE = 2048


def _argmin_body(z_ref, p_ref, out_ref, minval_ref, minidx_ref):
    k = pl.program_id(0)
    m = pl.program_id(1)
    z = z_ref[...]
    p = p_ref[...]
    z2 = jnp.sum(z * z, axis=1, keepdims=True)
    c2 = jnp.sum(p * p, axis=1)
    cross = lax.dot_general(z, p, (((1,), (1,)), ((), ())),
                            preferred_element_type=jnp.float32)
    d = jnp.maximum(z2 + c2[None, :] - 2.0 * cross, 0.0)
    lmin = jnp.min(d, axis=1, keepdims=True)
    iota = lax.broadcasted_iota(jnp.int32, d.shape, 1)
    larg = jnp.min(jnp.where(d == lmin, iota, 2**30), axis=1)
    lminv = lmin[:, 0]
    karg = larg + k * K_TILE

    @pl.when(k == 0)
    def _init():
        minval_ref[m, :] = lminv
        minidx_ref[m, :] = karg

    @pl.when(k > 0)
    def _update():
        better = lminv < minval_ref[m, :]
        minval_ref[m, :] = jnp.where(better, lminv, minval_ref[m, :])
        minidx_ref[m, :] = jnp.where(better, karg, minidx_ref[m, :])

    @pl.when(k == pl.num_programs(0) - 1)
    def _emit():
        out_ref[0, 0, :] = minidx_ref[m, :]


def _tc_argmin(z, protos):
    m_total, d_dim = z.shape
    k_total = protos.shape[0]
    nm = m_total // M_TILE
    nk = k_total // K_TILE
    return pl.pallas_call(
        _argmin_body,
        grid=(nk, nm),
        in_specs=[
            pl.BlockSpec((M_TILE, d_dim), lambda k, m: (m, 0)),
            pl.BlockSpec((K_TILE, d_dim), lambda k, m: (k, 0)),
        ],
        out_specs=pl.BlockSpec((1, 1, M_TILE), lambda k, m: (m, 0, 0)),
        out_shape=jax.ShapeDtypeStruct((nm, 1, M_TILE), jnp.int32),
        scratch_shapes=[
            pltpu.VMEM((nm, M_TILE), jnp.float32),
            pltpu.VMEM((nm, M_TILE), jnp.int32),
        ],
    )(z, protos)


_NC, _NS = 2, 16
_CHUNK = 128


def _make_sc_scatter(m_total, k_total, d_dim):
    dh = d_dim // _NC
    ppt = m_total // _NS
    n_chunks = ppt // _CHUNK
    k_rows = k_total // _NS
    mesh = plsc.VectorSubcoreMesh(core_axis_name="c", subcore_axis_name="s")

    @functools.partial(
        pl.kernel, mesh=mesh,
        out_type=[jax.ShapeDtypeStruct((k_total, d_dim), jnp.float32),
                  jax.ShapeDtypeStruct((k_total,), jnp.float32)],
        scratch_types=[
            pltpu.VMEM((n_chunks, _CHUNK), jnp.int32),
            pltpu.VMEM((_CHUNK, dh), jnp.float32),
            pltpu.VMEM((_CHUNK,), jnp.float32),
            pltpu.VMEM_SHARED((k_total, dh), jnp.float32),
            pltpu.VMEM_SHARED((k_total,), jnp.float32),
        ],
    )
    def scatter(z_hbm, idx_hbm, zeros2_hbm, zeros1_hbm, sums_hbm, cnt_hbm,
                idx_v, rows_v, ones_v, acc_sh, cnt_sh):
        c = lax.axis_index("c")
        s = lax.axis_index("s")
        pltpu.sync_copy(zeros2_hbm.at[pl.ds(s * k_rows, k_rows)],
                        acc_sh.at[pl.ds(s * k_rows, k_rows)])

        @pl.when(c == 0)
        def _zero_counts():
            pltpu.sync_copy(zeros1_hbm.at[pl.ds(s * k_rows, k_rows)],
                            cnt_sh.at[pl.ds(s * k_rows, k_rows)])

        pltpu.sync_copy(idx_hbm.at[pl.ds(s * n_chunks, n_chunks)], idx_v)
        for i in range(_CHUNK // 16):
            ones_v[pl.ds(i * 16, 16)] = jnp.full((16,), 1.0, jnp.float32)
        plsc.subcore_barrier()

        for j in range(n_chunks):
            pbase = s * ppt + j * _CHUNK
            pltpu.sync_copy(z_hbm.at[pl.ds(pbase, _CHUNK),
                                     pl.ds(c * dh, dh)], rows_v)
            pltpu.sync_copy(rows_v, acc_sh.at[idx_v.at[j]], add=True)

            @pl.when(c == 0)
            def _count():
                pltpu.sync_copy(ones_v, cnt_sh.at[idx_v.at[j]], add=True)

        plsc.subcore_barrier()
        pltpu.sync_copy(acc_sh.at[pl.ds(s * k_rows, k_rows)],
                        sums_hbm.at[pl.ds(s * k_rows, k_rows),
                                    pl.ds(c * dh, dh)])

        @pl.when(c == 0)
        def _emit_counts():
            pltpu.sync_copy(cnt_sh.at[pl.ds(s * k_rows, k_rows)],
                            cnt_hbm.at[pl.ds(s * k_rows, k_rows)])

    return scatter


def kernel(batch, protos):
    b, n, d_dim = batch.shape
    k_total = protos.shape[0]
    m_total = b * n
    z = batch.reshape(m_total, d_dim)
    closest = _tc_argmin(z, protos).reshape(b, n)
    idx2d = closest.reshape(m_total // _CHUNK, _CHUNK)
    zeros2 = jnp.zeros((k_total, d_dim // _NC), jnp.float32)
    zeros1 = jnp.zeros((k_total,), jnp.float32)
    sums, counts = _make_sc_scatter(m_total, k_total, d_dim)(
        z, idx2d, zeros2, zeros1)
    return closest, sums, counts
